# Initial kernel scaffold; baseline (speedup 1.0000x reference)
import jax
import jax.numpy as jnp
from jax.experimental import pallas as pl


def kernel(z, x, w_down, b_down, w_up, b_up):
    raise NotImplementedError("write your pallas kernel here")


# speedup vs baseline: 1.4685x; 1.4685x over previous
import jax
import jax.numpy as jnp
from jax import lax
from jax.experimental import pallas as pl
from jax.experimental.pallas import tpu as pltpu

_C = 384
_NS = 1024
_WS = 4
_WNH = 8
_NW = 64
_K = 16
_G = 24
_GD = 4 * _G
_WIDE = _WS * _C


def _body(z_ref, x_ref, wd_ref, bd_ref, wu_ref, bu_ref, out_ref, xe_ref):
    z = z_ref[0]
    zmax = jnp.max(z, axis=0, keepdims=True)
    zt = jnp.concatenate([zmax] * _WS, axis=1)
    xw = x_ref[0]
    xb = xw.astype(jnp.bfloat16).astype(jnp.float32)
    zb = zt.astype(jnp.bfloat16).astype(jnp.float32)
    rowsum = jnp.sum(xb * zb, axis=1, keepdims=True)

    wi = lax.broadcasted_iota(jnp.int32, (_NW, 256), 0)
    ji = lax.broadcasted_iota(jnp.int32, (_NW, 256), 1)
    q_sel = ((ji // 32 == wi // _WNH) & (ji % _WNH == wi % _WNH)).astype(
        jnp.float32)
    win = lax.dot_general(q_sel, rowsum, (((1,), (0,)), ((), ())),
                          preferred_element_type=jnp.float32,
                          precision=lax.Precision.HIGHEST)
    win = win * (1.0 / (_C * _WS * _WS))

    fidx = lax.broadcasted_iota(jnp.int32, (_NW, 1), 0)
    cur = win
    for k in range(_K):
        m = jnp.max(cur)
        w = jnp.min(jnp.where(cur == m, fidx, _NW))
        cur = jnp.where(fidx == w, -jnp.inf, cur)
        wr = w // _WNH
        wc = w - wr * _WNH
        for r in range(_WS):
            j = wr * 32 + r * _WNH + wc
            xe_ref[4 * k + r, :] = x_ref[0, pl.ds(j, 1), :][0]

    xe = xe_ref[...]
    wd = wd_ref[...]
    bd = bd_ref[...]
    wu = wu_ref[...]
    bu = bu_ref[...]

    xcs = [lax.slice(xe, (0, _C * c), (4 * _K, _C * (c + 1)))
           for c in range(_WS)]
    ts = [lax.dot_general(xc, wd, (((1,), (1,)), ((), ())),
                          preferred_element_type=jnp.float32) + bd
          for xc in xcs]

    qi = lax.broadcasted_iota(jnp.int32, (4 * _K, _GD), 0)
    li = lax.broadcasted_iota(jnp.int32, (4 * _K, _GD), 1)
    r_lt3 = (qi % _WS) < (_WS - 1)
    r_gt0 = (qi % _WS) > 0
    zeros = jnp.zeros((4 * _K, _GD), jnp.float32)
    zrow = jnp.zeros((1, _GD), jnp.float32)

    outs = []
    for c in range(_WS):
        t_c = ts[c]
        g0 = ts[c + 1] if c < _WS - 1 else zeros
        g1 = ts[c - 1] if c > 0 else zeros
        tdn = jnp.concatenate([t_c[1:], zrow], axis=0)
        tup = jnp.concatenate([zrow, t_c[:-1]], axis=0)
        g2 = jnp.where(r_lt3, tdn, 0.0)
        g3 = jnp.where(r_gt0, tup, 0.0)
        s_c = jnp.where(li < _G, g0,
                        jnp.where(li < 2 * _G, g1,
                                  jnp.where(li < 3 * _G, g2, g3)))
        su = lax.dot_general(s_c, wu, (((1,), (1,)), ((), ())),
                             preferred_element_type=jnp.float32)
        outs.append(xcs[c] + su + bu)
    out_ref[0] = jnp.concatenate(outs, axis=1)


def kernel(z, x, w_down, b_down, w_up, b_up):
    B = z.shape[0]
    x2 = x.reshape(B, _NS // _WS, _WIDE)
    bd = b_down.reshape(1, _GD)
    bu = b_up.reshape(1, _C)
    out = pl.pallas_call(
        _body,
        grid=(B,),
        in_specs=[
            pl.BlockSpec((1, z.shape[1], _C), lambda b: (b, 0, 0)),
            pl.BlockSpec((1, _NS // _WS, _WIDE), lambda b: (b, 0, 0)),
            pl.BlockSpec((_GD, _C), lambda b: (0, 0)),
            pl.BlockSpec((1, _GD), lambda b: (0, 0)),
            pl.BlockSpec((_C, _GD), lambda b: (0, 0)),
            pl.BlockSpec((1, _C), lambda b: (0, 0)),
        ],
        out_specs=pl.BlockSpec((1, 4 * _K, _WIDE), lambda b: (b, 0, 0)),
        out_shape=jax.ShapeDtypeStruct((B, 4 * _K, _WIDE), jnp.float32),
        scratch_shapes=[pltpu.VMEM((4 * _K, _WIDE), jnp.float32)],
    )(z, x2, w_down, bd, w_up, bu)
    return out.reshape(B, _K, _WS, _WS, _C).reshape(B, _K * _WS * _WS, _C)


# speedup vs baseline: 1.5995x; 1.0892x over previous
import jax
import jax.numpy as jnp
from jax import lax
from jax.experimental import pallas as pl
from jax.experimental.pallas import tpu as pltpu

_C = 384
_NS = 1024
_WS = 4
_WNH = 8
_NW = 64
_K = 16
_G = 24
_GD = 4 * _G
_WIDE = _WS * _C
_NB = 4


def _body(z_ref, x_ref, wd_ref, bd_ref, wu_ref, bu_ref, out_ref, xe_ref):
    nrow = 4 * _K
    wi = lax.broadcasted_iota(jnp.int32, (_NW, 256), 0)
    ji = lax.broadcasted_iota(jnp.int32, (_NW, 256), 1)
    q_sel = ((ji // 32 == wi // _WNH) & (ji % _WNH == wi % _WNH)).astype(
        jnp.float32)
    fidx = lax.broadcasted_iota(jnp.int32, (_NW, 1), 0)

    for i in range(_NB):
        z = z_ref[i]
        zmax = jnp.max(z, axis=0, keepdims=True)
        zt = jnp.concatenate([zmax] * _WS, axis=1)
        xw = x_ref[i]
        xb = xw.astype(jnp.bfloat16).astype(jnp.float32)
        zb = zt.astype(jnp.bfloat16).astype(jnp.float32)
        rowsum = jnp.sum(xb * zb, axis=1, keepdims=True)
        win = lax.dot_general(q_sel, rowsum, (((1,), (0,)), ((), ())),
                              preferred_element_type=jnp.float32,
                              precision=lax.Precision.HIGHEST)
        win = win * (1.0 / (_C * _WS * _WS))

        cur = win
        for k in range(_K):
            m = jnp.max(cur)
            w = jnp.min(jnp.where(cur == m, fidx, _NW))
            cur = jnp.where(fidx == w, -jnp.inf, cur)
            wr = w // _WNH
            wc = w - wr * _WNH
            for r in range(_WS):
                j = wr * 32 + r * _WNH + wc
                xe_ref[pl.ds(i * nrow + 4 * k + r, 1), :] = (
                    x_ref[i, pl.ds(j, 1), :])

    rows = _NB * nrow
    xe = xe_ref[...]
    wd = wd_ref[...]
    bd = bd_ref[...]
    wu = wu_ref[...]
    bu = bu_ref[...]

    xcs = [lax.slice(xe, (0, _C * c), (rows, _C * (c + 1)))
           for c in range(_WS)]
    t_full = lax.dot_general(jnp.concatenate(xcs, axis=0), wd,
                             (((1,), (1,)), ((), ())),
                             preferred_element_type=jnp.float32) + bd
    ts = [lax.slice(t_full, (rows * c, 0), (rows * (c + 1), _GD))
          for c in range(_WS)]

    qi = lax.broadcasted_iota(jnp.int32, (rows, _GD), 0)
    li = lax.broadcasted_iota(jnp.int32, (rows, _GD), 1)
    r_lt3 = (qi % _WS) < (_WS - 1)
    r_gt0 = (qi % _WS) > 0
    zeros = jnp.zeros((rows, _GD), jnp.float32)
    zrow = jnp.zeros((1, _GD), jnp.float32)

    scs = []
    for c in range(_WS):
        t_c = ts[c]
        g0 = ts[c + 1] if c < _WS - 1 else zeros
        g1 = ts[c - 1] if c > 0 else zeros
        tdn = jnp.concatenate([t_c[1:], zrow], axis=0)
        tup = jnp.concatenate([zrow, t_c[:-1]], axis=0)
        g2 = jnp.where(r_lt3, tdn, 0.0)
        g3 = jnp.where(r_gt0, tup, 0.0)
        scs.append(jnp.where(li < _G, g0,
                             jnp.where(li < 2 * _G, g1,
                                       jnp.where(li < 3 * _G, g2, g3))))
    su_full = lax.dot_general(jnp.concatenate(scs, axis=0), wu,
                              (((1,), (1,)), ((), ())),
                              preferred_element_type=jnp.float32)
    outs = [xcs[c]
            + lax.slice(su_full, (rows * c, 0), (rows * (c + 1), _C))
            + bu
            for c in range(_WS)]
    out_wide = jnp.concatenate(outs, axis=1)
    for i in range(_NB):
        out_ref[i] = out_wide[i * nrow:(i + 1) * nrow]


def kernel(z, x, w_down, b_down, w_up, b_up):
    B = z.shape[0]
    x2 = x.reshape(B, _NS // _WS, _WIDE)
    bd = b_down.reshape(1, _GD)
    bu = b_up.reshape(1, _C)
    out = pl.pallas_call(
        _body,
        grid=(B // _NB,),
        in_specs=[
            pl.BlockSpec((_NB, z.shape[1], _C), lambda b: (b, 0, 0)),
            pl.BlockSpec((_NB, _NS // _WS, _WIDE), lambda b: (b, 0, 0)),
            pl.BlockSpec((_GD, _C), lambda b: (0, 0)),
            pl.BlockSpec((1, _GD), lambda b: (0, 0)),
            pl.BlockSpec((_C, _GD), lambda b: (0, 0)),
            pl.BlockSpec((1, _C), lambda b: (0, 0)),
        ],
        out_specs=pl.BlockSpec((_NB, 4 * _K, _WIDE), lambda b: (b, 0, 0)),
        out_shape=jax.ShapeDtypeStruct((B, 4 * _K, _WIDE), jnp.float32),
        scratch_shapes=[pltpu.VMEM((_NB * 4 * _K, _WIDE), jnp.float32)],
    )(z, x2, w_down, bd, w_up, bu)
    return out.reshape(B, _K, _WS, _WS, _C).reshape(B, _K * _WS * _WS, _C)


# speedup vs baseline: 3.7759x; 2.3607x over previous
import jax
import jax.numpy as jnp
from jax import lax
from jax.experimental import pallas as pl
from jax.experimental.pallas import tpu as pltpu

_C = 384
_NS = 1024
_WS = 4
_WNH = 8
_NW = 64
_K = 16
_G = 24
_GD = 4 * _G
_WIDE = _WS * _C
_NB = 4


def _body(z_ref, x_ref, wd_ref, bd_ref, wu_ref, bu_ref, out_ref, xe_ref):
    nrow = 4 * _K
    wi = lax.broadcasted_iota(jnp.int32, (_NW, 256), 0)
    ji = lax.broadcasted_iota(jnp.int32, (_NW, 256), 1)
    q_sel = ((ji // 32 == wi // _WNH) & (ji % _WNH == wi % _WNH)).astype(
        jnp.float32)
    eye = (lax.broadcasted_iota(jnp.int32, (_NW, _NW), 0)
           == lax.broadcasted_iota(jnp.int32, (_NW, _NW), 1)).astype(
        jnp.float32)
    wi64 = lax.broadcasted_iota(jnp.int32, (_NW, _NW), 0)
    ji64 = lax.broadcasted_iota(jnp.int32, (_NW, _NW), 1)
    fidx = lax.broadcasted_iota(jnp.int32, (_NW, 1), 0)
    rhsT = (((1,), (1,)), ((), ()))

    for i in range(_NB):
        z = z_ref[i]
        zmax = jnp.max(z, axis=0, keepdims=True)
        zt = jnp.concatenate([zmax] * _WS, axis=1)
        xw = x_ref[i]
        rowsum = lax.dot_general(zt, xw, rhsT,
                                 preferred_element_type=jnp.float32)
        win_row = lax.dot_general(rowsum, q_sel, rhsT,
                                  preferred_element_type=jnp.float32,
                                  precision=lax.Precision.HIGHEST)
        win_col = lax.dot_general(eye, win_row, rhsT,
                                  preferred_element_type=jnp.float32,
                                  precision=lax.Precision.HIGHEST)
        vj = jnp.broadcast_to(win_row, (_NW, _NW))
        vw = jnp.broadcast_to(win_col, (_NW, _NW))
        beats = (vj > vw) | ((vj == vw) & (ji64 < wi64))
        rank = jnp.sum(beats.astype(jnp.int32), axis=1, keepdims=True)
        for k in range(_K):
            w = jnp.sum(jnp.where(rank == k, fidx, 0))
            wr = w // _WNH
            wc = w - wr * _WNH
            for r in range(_WS):
                j = wr * 32 + r * _WNH + wc
                xe_ref[pl.ds(i * nrow + 4 * k + r, 1), :] = (
                    x_ref[i, pl.ds(j, 1), :])

    rows = _NB * nrow
    xe = xe_ref[...]
    wd = wd_ref[...]
    bd = bd_ref[...]
    wu = wu_ref[...]
    bu = bu_ref[...]

    xcs = [lax.slice(xe, (0, _C * c), (rows, _C * (c + 1)))
           for c in range(_WS)]
    t_full = lax.dot_general(jnp.concatenate(xcs, axis=0), wd,
                             (((1,), (1,)), ((), ())),
                             preferred_element_type=jnp.float32) + bd
    ts = [lax.slice(t_full, (rows * c, 0), (rows * (c + 1), _GD))
          for c in range(_WS)]

    qi = lax.broadcasted_iota(jnp.int32, (rows, _GD), 0)
    li = lax.broadcasted_iota(jnp.int32, (rows, _GD), 1)
    r_lt3 = (qi % _WS) < (_WS - 1)
    r_gt0 = (qi % _WS) > 0
    zeros = jnp.zeros((rows, _GD), jnp.float32)
    zrow = jnp.zeros((1, _GD), jnp.float32)

    scs = []
    for c in range(_WS):
        t_c = ts[c]
        g0 = ts[c + 1] if c < _WS - 1 else zeros
        g1 = ts[c - 1] if c > 0 else zeros
        tdn = jnp.concatenate([t_c[1:], zrow], axis=0)
        tup = jnp.concatenate([zrow, t_c[:-1]], axis=0)
        g2 = jnp.where(r_lt3, tdn, 0.0)
        g3 = jnp.where(r_gt0, tup, 0.0)
        scs.append(jnp.where(li < _G, g0,
                             jnp.where(li < 2 * _G, g1,
                                       jnp.where(li < 3 * _G, g2, g3))))
    su_full = lax.dot_general(jnp.concatenate(scs, axis=0), wu,
                              (((1,), (1,)), ((), ())),
                              preferred_element_type=jnp.float32)
    outs = [xcs[c]
            + lax.slice(su_full, (rows * c, 0), (rows * (c + 1), _C))
            + bu
            for c in range(_WS)]
    out_wide = jnp.concatenate(outs, axis=1)
    for i in range(_NB):
        out_ref[i] = out_wide[i * nrow:(i + 1) * nrow]


def kernel(z, x, w_down, b_down, w_up, b_up):
    B = z.shape[0]
    x2 = x.reshape(B, _NS // _WS, _WIDE)
    bd = b_down.reshape(1, _GD)
    bu = b_up.reshape(1, _C)
    out = pl.pallas_call(
        _body,
        grid=(B // _NB,),
        in_specs=[
            pl.BlockSpec((_NB, z.shape[1], _C), lambda b: (b, 0, 0)),
            pl.BlockSpec((_NB, _NS // _WS, _WIDE), lambda b: (b, 0, 0)),
            pl.BlockSpec((_GD, _C), lambda b: (0, 0)),
            pl.BlockSpec((1, _GD), lambda b: (0, 0)),
            pl.BlockSpec((_C, _GD), lambda b: (0, 0)),
            pl.BlockSpec((1, _C), lambda b: (0, 0)),
        ],
        out_specs=pl.BlockSpec((_NB, 4 * _K, _WIDE), lambda b: (b, 0, 0)),
        out_shape=jax.ShapeDtypeStruct((B, 4 * _K, _WIDE), jnp.float32),
        scratch_shapes=[pltpu.VMEM((_NB * 4 * _K, _WIDE), jnp.float32)],
    )(z, x2, w_down, bd, w_up, bu)
    return out.reshape(B, _K, _WS, _WS, _C).reshape(B, _K * _WS * _WS, _C)


# speedup vs baseline: 10.0745x; 2.6681x over previous
import jax
import jax.numpy as jnp
from jax import lax
from jax.experimental import pallas as pl
from jax.experimental.pallas import tpu as pltpu

_C = 384
_NS = 1024
_WS = 4
_WNH = 8
_NW = 64
_K = 16
_G = 24
_GD = 4 * _G
_NT = _K * _WS * _WS
_NB = 4

_RHS_T = (((1,), (1,)), ((), ()))


def _body(z_ref, x_ref, wd_ref, bd_ref, wu_ref, bu_ref, out_ref, xe_ref):
    ti = lax.broadcasted_iota(jnp.int32, (_NW, _NS), 1)
    wi = lax.broadcasted_iota(jnp.int32, (_NW, _NS), 0)
    q_sel = ((ti // 128 == wi // _WNH)
             & ((ti % 32) // _WS == wi % _WNH)).astype(jnp.float32)
    eye = (lax.broadcasted_iota(jnp.int32, (_NW, _NW), 0)
           == lax.broadcasted_iota(jnp.int32, (_NW, _NW), 1)).astype(
        jnp.float32)
    wi64 = lax.broadcasted_iota(jnp.int32, (_NW, _NW), 0)
    ji64 = lax.broadcasted_iota(jnp.int32, (_NW, _NW), 1)
    fidx = lax.broadcasted_iota(jnp.int32, (_NW, 1), 0)

    for i in range(_NB):
        z = z_ref[i]
        zmax = jnp.max(z, axis=0, keepdims=True)
        sim = lax.dot_general(zmax, x_ref[i], _RHS_T,
                              preferred_element_type=jnp.float32)
        win_row = lax.dot_general(sim, q_sel, _RHS_T,
                                  preferred_element_type=jnp.float32,
                                  precision=lax.Precision.HIGHEST)
        win_col = lax.dot_general(eye, win_row, _RHS_T,
                                  preferred_element_type=jnp.float32,
                                  precision=lax.Precision.HIGHEST)
        vj = jnp.broadcast_to(win_row, (_NW, _NW))
        vw = jnp.broadcast_to(win_col, (_NW, _NW))
        beats = (vj > vw) | ((vj == vw) & (ji64 < wi64))
        rank = jnp.sum(beats.astype(jnp.int32), axis=1, keepdims=True)
        for k in range(_K):
            w = jnp.sum(jnp.where(rank == k, fidx, 0))
            wr = w // _WNH
            wc = w - wr * _WNH
            for r in range(_WS):
                t0 = wr * 128 + r * 32 + wc * _WS
                row0 = i * _NT + _K * k + _WS * r
                for c in range(_WS):
                    xe_ref[pl.ds(row0 + c, 1), :] = (
                        x_ref[i, pl.ds(t0 + c, 1), :])

    rows = _NB * _NT
    xe = xe_ref[...]
    wd = wd_ref[...]
    bd = bd_ref[...]
    wu = wu_ref[...]
    bu = bu_ref[...]

    t = lax.dot_general(xe, wd, _RHS_T,
                        preferred_element_type=jnp.float32) + bd

    qi = lax.broadcasted_iota(jnp.int32, (rows, _GD), 0)
    li = lax.broadcasted_iota(jnp.int32, (rows, _GD), 1)
    c_tok = qi % _WS
    r_tok = (qi // _WS) % _WS
    z1 = jnp.zeros((1, _GD), jnp.float32)
    z4 = jnp.zeros((_WS, _GD), jnp.float32)
    tm1 = jnp.concatenate([t[1:], z1], axis=0)
    tp1 = jnp.concatenate([z1, t[:-1]], axis=0)
    tm4 = jnp.concatenate([t[_WS:], z4], axis=0)
    tp4 = jnp.concatenate([z4, t[:-_WS]], axis=0)
    g0 = jnp.where(c_tok < _WS - 1, tm1, 0.0)
    g1 = jnp.where(c_tok > 0, tp1, 0.0)
    g2 = jnp.where(r_tok < _WS - 1, tm4, 0.0)
    g3 = jnp.where(r_tok > 0, tp4, 0.0)
    s = jnp.where(li < _G, g0,
                  jnp.where(li < 2 * _G, g1,
                            jnp.where(li < 3 * _G, g2, g3)))
    su = lax.dot_general(s, wu, _RHS_T,
                         preferred_element_type=jnp.float32)
    out = xe + su + bu
    for i in range(_NB):
        out_ref[i] = out[i * _NT:(i + 1) * _NT]


def kernel(z, x, w_down, b_down, w_up, b_up):
    B = z.shape[0]
    bd = b_down.reshape(1, _GD)
    bu = b_up.reshape(1, _C)
    return pl.pallas_call(
        _body,
        grid=(B // _NB,),
        in_specs=[
            pl.BlockSpec((_NB, z.shape[1], _C), lambda b: (b, 0, 0)),
            pl.BlockSpec((_NB, _NS, _C), lambda b: (b, 0, 0)),
            pl.BlockSpec((_GD, _C), lambda b: (0, 0)),
            pl.BlockSpec((1, _GD), lambda b: (0, 0)),
            pl.BlockSpec((_C, _GD), lambda b: (0, 0)),
            pl.BlockSpec((1, _C), lambda b: (0, 0)),
        ],
        out_specs=pl.BlockSpec((_NB, _NT, _C), lambda b: (b, 0, 0)),
        out_shape=jax.ShapeDtypeStruct((B, _NT, _C), jnp.float32),
        scratch_shapes=[pltpu.VMEM((_NB * _NT, _C), jnp.float32)],
    )(z, x, w_down, bd, w_up, bu)


# speedup vs baseline: 11.2776x; 1.1194x over previous
import jax
import jax.numpy as jnp
from jax import lax
from jax.experimental import pallas as pl
from jax.experimental.pallas import tpu as pltpu

_C = 384
_NS = 1024
_WS = 4
_WNH = 8
_NW = 64
_K = 16
_G = 24
_GD = 4 * _G
_NT = _K * _WS * _WS
_NB = 4

_RHS_T = (((1,), (1,)), ((), ()))


def _body(z_ref, x_ref, wd_ref, bd_ref, wu_ref, bu_ref, out_ref, xe_ref):
    ti = lax.broadcasted_iota(jnp.int32, (_NW, _NS), 1)
    wi = lax.broadcasted_iota(jnp.int32, (_NW, _NS), 0)
    q_sel = ((ti // 128 == wi // _WNH)
             & ((ti % 32) // _WS == wi % _WNH)).astype(jnp.float32)
    eye = (lax.broadcasted_iota(jnp.int32, (_NW, _NW), 0)
           == lax.broadcasted_iota(jnp.int32, (_NW, _NW), 1)).astype(
        jnp.float32)
    wi64 = lax.broadcasted_iota(jnp.int32, (_NW, _NW), 0)
    ji64 = lax.broadcasted_iota(jnp.int32, (_NW, _NW), 1)
    fidx = lax.broadcasted_iota(jnp.int32, (_NW, 1), 0)

    for i in range(_NB):
        z = z_ref[i]
        zmax = jnp.max(z, axis=0, keepdims=True)
        sim = lax.dot_general(zmax, x_ref[i], _RHS_T,
                              preferred_element_type=jnp.float32)
        win_row = lax.dot_general(sim, q_sel, _RHS_T,
                                  preferred_element_type=jnp.float32,
                                  precision=lax.Precision.HIGHEST)
        win_col = lax.dot_general(eye, win_row, _RHS_T,
                                  preferred_element_type=jnp.float32,
                                  precision=lax.Precision.HIGHEST)
        vj = jnp.broadcast_to(win_row, (_NW, _NW))
        vw = jnp.broadcast_to(win_col, (_NW, _NW))
        beats = (vj > vw) | ((vj == vw) & (ji64 < wi64))
        rank = jnp.sum(beats.astype(jnp.int32), axis=1, keepdims=True)
        for k in range(_K):
            w = jnp.sum(jnp.where(rank == k, fidx, 0))
            wr = w // _WNH
            base = wr * 128 + (w - wr * _WNH) * _WS
            row0 = i * _NT + _K * k
            for r in range(_WS):
                for c in range(_WS):
                    xe_ref[pl.ds(row0 + _WS * r + c, 1), :] = (
                        x_ref[i, pl.ds(base + 32 * r + c, 1), :])

    rows = _NB * _NT
    xe = xe_ref[...]
    wd = wd_ref[...]
    bd = bd_ref[...]
    wu = wu_ref[...]
    bu = bu_ref[...]

    t = lax.dot_general(xe, wd, _RHS_T,
                        preferred_element_type=jnp.float32) + bd

    qi = lax.broadcasted_iota(jnp.int32, (rows, _GD), 0)
    li = lax.broadcasted_iota(jnp.int32, (rows, _GD), 1)
    c_tok = qi % _WS
    r_tok = (qi // _WS) % _WS
    z1 = jnp.zeros((1, _GD), jnp.float32)
    z4 = jnp.zeros((_WS, _GD), jnp.float32)
    tm1 = jnp.concatenate([t[1:], z1], axis=0)
    tp1 = jnp.concatenate([z1, t[:-1]], axis=0)
    tm4 = jnp.concatenate([t[_WS:], z4], axis=0)
    tp4 = jnp.concatenate([z4, t[:-_WS]], axis=0)
    g0 = jnp.where(c_tok < _WS - 1, tm1, 0.0)
    g1 = jnp.where(c_tok > 0, tp1, 0.0)
    g2 = jnp.where(r_tok < _WS - 1, tm4, 0.0)
    g3 = jnp.where(r_tok > 0, tp4, 0.0)
    s = jnp.where(li < _G, g0,
                  jnp.where(li < 2 * _G, g1,
                            jnp.where(li < 3 * _G, g2, g3)))
    su = lax.dot_general(s, wu, _RHS_T,
                         preferred_element_type=jnp.float32)
    out = xe + su + bu
    for i in range(_NB):
        out_ref[i] = out[i * _NT:(i + 1) * _NT]


def kernel(z, x, w_down, b_down, w_up, b_up):
    B = z.shape[0]
    bd = b_down.reshape(1, _GD)
    bu = b_up.reshape(1, _C)
    return pl.pallas_call(
        _body,
        grid=(B // _NB,),
        in_specs=[
            pl.BlockSpec((_NB, z.shape[1], _C), lambda b: (b, 0, 0)),
            pl.BlockSpec((_NB, _NS, _C), lambda b: (b, 0, 0)),
            pl.BlockSpec((_GD, _C), lambda b: (0, 0)),
            pl.BlockSpec((1, _GD), lambda b: (0, 0)),
            pl.BlockSpec((_C, _GD), lambda b: (0, 0)),
            pl.BlockSpec((1, _C), lambda b: (0, 0)),
        ],
        out_specs=pl.BlockSpec((_NB, _NT, _C), lambda b: (b, 0, 0)),
        out_shape=jax.ShapeDtypeStruct((B, _NT, _C), jnp.float32),
        scratch_shapes=[pltpu.VMEM((_NB * _NT, _C), jnp.float32)],
    )(z, x, w_down, bd, w_up, bu)


# speedup vs baseline: 12.0270x; 1.0665x over previous
import jax
import jax.numpy as jnp
from jax import lax
from jax.experimental import pallas as pl
from jax.experimental.pallas import tpu as pltpu

_C = 384
_NS = 1024
_WS = 4
_WNH = 8
_NW = 64
_K = 16
_G = 24
_GD = 4 * _G
_NT = _K * _WS * _WS
_NB = 8

_RHS_T = (((1,), (1,)), ((), ()))


def _body(z_ref, x_ref, wd_ref, bd_ref, wu_ref, bu_ref, out_ref, xe_ref):
    ti = lax.broadcasted_iota(jnp.int32, (_NW, _NS), 1)
    wi = lax.broadcasted_iota(jnp.int32, (_NW, _NS), 0)
    q_sel = ((ti // 128 == wi // _WNH)
             & ((ti % 32) // _WS == wi % _WNH)).astype(jnp.float32)
    eye = (lax.broadcasted_iota(jnp.int32, (_NW, _NW), 0)
           == lax.broadcasted_iota(jnp.int32, (_NW, _NW), 1)).astype(
        jnp.float32)
    wi64 = lax.broadcasted_iota(jnp.int32, (_NW, _NW), 0)
    ji64 = lax.broadcasted_iota(jnp.int32, (_NW, _NW), 1)
    fidx = lax.broadcasted_iota(jnp.int32, (_NW, 1), 0)

    for i in range(_NB):
        z = z_ref[i]
        zmax = jnp.max(z, axis=0, keepdims=True)
        sim = lax.dot_general(zmax, x_ref[i], _RHS_T,
                              preferred_element_type=jnp.float32)
        win_row = lax.dot_general(sim, q_sel, _RHS_T,
                                  preferred_element_type=jnp.float32,
                                  precision=lax.Precision.HIGHEST)
        win_col = lax.dot_general(eye, win_row, _RHS_T,
                                  preferred_element_type=jnp.float32,
                                  precision=lax.Precision.HIGHEST)
        vj = jnp.broadcast_to(win_row, (_NW, _NW))
        vw = jnp.broadcast_to(win_col, (_NW, _NW))
        beats = (vj > vw) | ((vj == vw) & (ji64 < wi64))
        rank = jnp.sum(beats.astype(jnp.int32), axis=1, keepdims=True)
        for k in range(_K):
            w = jnp.sum(jnp.where(rank == k, fidx, 0))
            wr = w // _WNH
            base = wr * 128 + (w - wr * _WNH) * _WS
            row0 = i * _NT + _K * k
            for r in range(_WS):
                for c in range(_WS):
                    xe_ref[pl.ds(row0 + _WS * r + c, 1), :] = (
                        x_ref[i, pl.ds(base + 32 * r + c, 1), :])

    rows = _NB * _NT
    xe = xe_ref[...]
    wd = wd_ref[...]
    bd = bd_ref[...]
    wu = wu_ref[...]
    bu = bu_ref[...]

    t = lax.dot_general(xe, wd, _RHS_T,
                        preferred_element_type=jnp.float32) + bd

    qi = lax.broadcasted_iota(jnp.int32, (rows, _GD), 0)
    li = lax.broadcasted_iota(jnp.int32, (rows, _GD), 1)
    c_tok = qi % _WS
    r_tok = (qi // _WS) % _WS
    z1 = jnp.zeros((1, _GD), jnp.float32)
    z4 = jnp.zeros((_WS, _GD), jnp.float32)
    tm1 = jnp.concatenate([t[1:], z1], axis=0)
    tp1 = jnp.concatenate([z1, t[:-1]], axis=0)
    tm4 = jnp.concatenate([t[_WS:], z4], axis=0)
    tp4 = jnp.concatenate([z4, t[:-_WS]], axis=0)
    g0 = jnp.where(c_tok < _WS - 1, tm1, 0.0)
    g1 = jnp.where(c_tok > 0, tp1, 0.0)
    g2 = jnp.where(r_tok < _WS - 1, tm4, 0.0)
    g3 = jnp.where(r_tok > 0, tp4, 0.0)
    s = jnp.where(li < _G, g0,
                  jnp.where(li < 2 * _G, g1,
                            jnp.where(li < 3 * _G, g2, g3)))
    su = lax.dot_general(s, wu, _RHS_T,
                         preferred_element_type=jnp.float32)
    out = xe + su + bu
    for i in range(_NB):
        out_ref[i] = out[i * _NT:(i + 1) * _NT]


def kernel(z, x, w_down, b_down, w_up, b_up):
    B = z.shape[0]
    bd = b_down.reshape(1, _GD)
    bu = b_up.reshape(1, _C)
    return pl.pallas_call(
        _body,
        grid=(B // _NB,),
        in_specs=[
            pl.BlockSpec((_NB, z.shape[1], _C), lambda b: (b, 0, 0)),
            pl.BlockSpec((_NB, _NS, _C), lambda b: (b, 0, 0)),
            pl.BlockSpec((_GD, _C), lambda b: (0, 0)),
            pl.BlockSpec((1, _GD), lambda b: (0, 0)),
            pl.BlockSpec((_C, _GD), lambda b: (0, 0)),
            pl.BlockSpec((1, _C), lambda b: (0, 0)),
        ],
        out_specs=pl.BlockSpec((_NB, _NT, _C), lambda b: (b, 0, 0)),
        out_shape=jax.ShapeDtypeStruct((B, _NT, _C), jnp.float32),
        scratch_shapes=[pltpu.VMEM((_NB * _NT, _C), jnp.float32)],
    )(z, x, w_down, bd, w_up, bu)
